# baseline (speedup 1.0000x reference)
import jax
import jax.numpy as jnp
from jax.experimental import pallas as pl
from jax.experimental.pallas import tpu as pltpu

_BN = 1024
_KC = 512
_LANES = 128


def _loss_kernel(e_ref, c_ref, out_ref, cblk_ref, csq_ref):
    i = pl.program_id(0)

    @pl.when(i == 0)
    def _build():
        c = c_ref[...]
        z = jnp.zeros_like(c)
        blk = jnp.concatenate(
            [jnp.concatenate([c, z], axis=1),
             jnp.concatenate([z, c], axis=1)], axis=0)
        cblk_ref[...] = (-2.0 * blk).astype(jnp.bfloat16)
        sq = cblk_ref[...] * cblk_ref[...]
        ones_row = jnp.ones((1, 2 * c.shape[1]), jnp.bfloat16)
        csq_ref[...] = 0.25 * jax.lax.dot_general(
            ones_row, sq, (((1,), (1,)), ((), ())),
            preferred_element_type=jnp.float32)

    e = e_ref[...]
    ebf = e.astype(jnp.bfloat16)
    cblk = cblk_ref[...]
    csq = csq_ref[...]
    k2 = cblk.shape[0]

    m_half = [None, None]
    for j in range(k2 // _KC):
        cj = cblk[j * _KC:(j + 1) * _KC, :]
        pj = jax.lax.dot_general(
            ebf, cj, (((1,), (1,)), ((), ())),
            preferred_element_type=jnp.float32)
        dj = pj + csq[:, j * _KC:(j + 1) * _KC]
        mj = dj[:, 0:_LANES]
        for t in range(1, _KC // _LANES):
            mj = jnp.minimum(mj, dj[:, t * _LANES:(t + 1) * _LANES])
        h = (j * _KC) // (k2 // 2)
        m_half[h] = mj if m_half[h] is None else jnp.minimum(m_half[h], mj)
    m0, m1 = m_half
    m_row = (jnp.min(m0, axis=1, keepdims=True)
             + jnp.min(m1, axis=1, keepdims=True))
    e_sq = jnp.sum(e * e, axis=1, keepdims=True)
    partial = jnp.sum(m_row + e_sq).reshape(1, 1)

    @pl.when(i == 0)
    def _init():
        out_ref[...] = jnp.zeros_like(out_ref)

    out_ref[...] += partial


def kernel(embedded, centers):
    n, d = embedded.shape
    k = centers.shape[0]
    e2 = embedded.reshape(n // 2, 2 * d)
    grid = (n // 2) // _BN
    total = pl.pallas_call(
        _loss_kernel,
        grid=(grid,),
        in_specs=[
            pl.BlockSpec((_BN, 2 * d), lambda i: (i, 0)),
            pl.BlockSpec((k, d), lambda i: (0, 0)),
        ],
        out_specs=pl.BlockSpec((1, 1), lambda i: (0, 0)),
        out_shape=jax.ShapeDtypeStruct((1, 1), jnp.float32),
        scratch_shapes=[
            pltpu.VMEM((2 * k, 2 * d), jnp.bfloat16),
            pltpu.VMEM((1, 2 * k), jnp.float32),
        ],
    )(e2, centers)
    return total[0, 0] / n


# speedup vs baseline: 2.1734x; 2.1734x over previous
import jax
import jax.numpy as jnp
from jax.experimental import pallas as pl
from jax.experimental.pallas import tpu as pltpu

_BN = 2048
_KC = 256
_SUB = 8


def _loss_kernel(et_ref, c_ref, out_ref, caug_ref):
    i = pl.program_id(0)

    @pl.when(i == 0)
    def _build():
        c = c_ref[...]
        c_sq = jnp.sum(c * c, axis=1, keepdims=True)
        hi = c_sq.astype(jnp.bfloat16)
        lo = (c_sq - hi.astype(jnp.float32)).astype(jnp.bfloat16)
        caug_ref[...] = jnp.concatenate(
            [(-2.0 * c).astype(jnp.bfloat16), hi, lo], axis=1)

    et = et_ref[...]
    bn = et.shape[1]
    et_aug = jnp.concatenate(
        [et.astype(jnp.bfloat16),
         jnp.ones((2, bn), jnp.bfloat16)], axis=0)
    c_aug = caug_ref[...]
    k = c_aug.shape[0]

    m_acc = None
    for j in range(k // _KC):
        cj = c_aug[j * _KC:(j + 1) * _KC, :]
        pj = jax.lax.dot_general(
            cj, et_aug, (((1,), (0,)), ((), ())),
            preferred_element_type=jnp.float32)
        mj = pj[0:_SUB, :]
        for t in range(1, _KC // _SUB):
            mj = jnp.minimum(mj, pj[t * _SUB:(t + 1) * _SUB, :])
        m_acc = mj if m_acc is None else jnp.minimum(m_acc, mj)
    m_col = jnp.min(m_acc, axis=0, keepdims=True)
    e_sq = jnp.sum(et * et, axis=0, keepdims=True)
    partial = jnp.sum(m_col + e_sq).reshape(1, 1)

    @pl.when(i == 0)
    def _init():
        out_ref[...] = jnp.zeros_like(out_ref)

    out_ref[...] += partial


def kernel(embedded, centers):
    n, d = embedded.shape
    k = centers.shape[0]
    et = embedded.T
    grid = n // _BN
    total = pl.pallas_call(
        _loss_kernel,
        grid=(grid,),
        in_specs=[
            pl.BlockSpec((d, _BN), lambda i: (0, i)),
            pl.BlockSpec((k, d), lambda i: (0, 0)),
        ],
        out_specs=pl.BlockSpec((1, 1), lambda i: (0, 0)),
        out_shape=jax.ShapeDtypeStruct((1, 1), jnp.float32),
        scratch_shapes=[pltpu.VMEM((k, d + 2), jnp.bfloat16)],
    )(et, centers)
    return total[0, 0] / n


# speedup vs baseline: 2.5982x; 1.1955x over previous
import jax
import jax.numpy as jnp
from jax.experimental import pallas as pl
from jax.experimental.pallas import tpu as pltpu

_BN = 16384
_KC = 256
_SUB = 8


def _loss_kernel(et_ref, c_ref, out_ref, caug_ref):
    i = pl.program_id(0)

    @pl.when(i == 0)
    def _build():
        c = c_ref[...]
        c_sq = jnp.sum(c * c, axis=1, keepdims=True)
        hi = c_sq.astype(jnp.bfloat16)
        lo = (c_sq - hi.astype(jnp.float32)).astype(jnp.bfloat16)
        caug_ref[...] = jnp.concatenate(
            [(-2.0 * c).astype(jnp.bfloat16), hi, lo], axis=1)

    et = et_ref[...]
    bn = et.shape[1]
    et_aug = jnp.concatenate(
        [et.astype(jnp.bfloat16),
         jnp.ones((2, bn), jnp.bfloat16)], axis=0)
    c_aug = caug_ref[...]
    k = c_aug.shape[0]

    m_acc = None
    for j in range(k // _KC):
        cj = c_aug[j * _KC:(j + 1) * _KC, :]
        pj = jax.lax.dot_general(
            cj, et_aug, (((1,), (0,)), ((), ())),
            preferred_element_type=jnp.float32)
        tiles = [pj[t * _SUB:(t + 1) * _SUB, :] for t in range(_KC // _SUB)]
        while len(tiles) > 1:
            tiles = [jnp.minimum(tiles[t], tiles[t + 1])
                     for t in range(0, len(tiles) - 1, 2)] + (
                         [tiles[-1]] if len(tiles) % 2 else [])
        mj = tiles[0]
        m_acc = mj if m_acc is None else jnp.minimum(m_acc, mj)
    m_col = jnp.min(m_acc, axis=0, keepdims=True)
    e_sq = jnp.sum(et * et, axis=0, keepdims=True)
    partial = jnp.sum(m_col + e_sq).reshape(1, 1)

    @pl.when(i == 0)
    def _init():
        out_ref[...] = jnp.zeros_like(out_ref)

    out_ref[...] += partial


def kernel(embedded, centers):
    n, d = embedded.shape
    k = centers.shape[0]
    et = embedded.T
    grid = n // _BN
    total = pl.pallas_call(
        _loss_kernel,
        grid=(grid,),
        in_specs=[
            pl.BlockSpec((d, _BN), lambda i: (0, i)),
            pl.BlockSpec((k, d), lambda i: (0, 0)),
        ],
        out_specs=pl.BlockSpec((1, 1), lambda i: (0, 0)),
        out_shape=jax.ShapeDtypeStruct((1, 1), jnp.float32),
        scratch_shapes=[pltpu.VMEM((k, d + 2), jnp.bfloat16)],
    )(et, centers)
    return total[0, 0] / n
